# Initial kernel scaffold; baseline (speedup 1.0000x reference)
import jax
import jax.numpy as jnp
from jax.experimental import pallas as pl


def kernel(feat, edge_index, W1, b1, Wc1, bc1, gc1, bec1, Wc2, bc2, gc2, bec2, Wm, bm, gm, bem, Wo, bo):
    raise NotImplementedError("write your pallas kernel here")


# speedup vs baseline: 8.0068x; 8.0068x over previous
import functools

import jax
import jax.numpy as jnp
from jax import lax
from jax.experimental import pallas as pl
from jax.experimental.pallas import tpu as pltpu
from jax.experimental.pallas import tpu_sc as plsc

N = 50000
E = 800000
EPS = 1e-5
NROWS = E // 128
NPAD = 50176
RPT = NPAD // 16
ZR = RPT // 8
BN = 1000
GRID = N // BN

_mesh = plsc.VectorSubcoreMesh(core_axis_name="c", subcore_axis_name="s")
_sc_params = pltpu.CompilerParams(use_tc_tiling_on_sc=False)


@functools.partial(
    pl.kernel,
    mesh=_mesh,
    out_type=[
        jax.ShapeDtypeStruct((2, NPAD, 16), jnp.float32),
        jax.ShapeDtypeStruct((2, 16, 16), jnp.float32),
    ],
    scratch_types=[
        pltpu.VMEM((8, 128), jnp.int32),
        pltpu.VMEM((8, 128), jnp.int32),
        pltpu.VMEM((1024, 16), jnp.float32),
        pltpu.VMEM((1024, 16), jnp.float32),
        pltpu.VMEM((1024, 16), jnp.float32),
        pltpu.VMEM((16,), jnp.float32),
        pltpu.VMEM((ZR, 16), jnp.float32),
        pltpu.VMEM_SHARED((NPAD, 16), jnp.float32),
        pltpu.SemaphoreType.DMA,
    ],
    compiler_params=_sc_params,
)
def _edge_sc(a_hbm, b_hbm, src_hbm, dst_hbm, psum_out, s2_out,
             sidx, didx, a_v, b_v, o_v, s2_v, zbuf, shared, gsem):
    c = lax.axis_index("c")
    s = lax.axis_index("s")
    zero16 = jnp.zeros((16,), jnp.float32)
    ah = a_hbm.at[c]
    bh = b_hbm.at[c]

    def _zb(r, carry):
        zbuf[r, 0:16] = zero16
        return carry
    lax.fori_loop(0, ZR, _zb, 0)
    for t in range(8):
        pltpu.sync_copy(zbuf, shared.at[pl.ds(s * RPT + t * ZR, ZR)])
    s2_v[0:16] = zero16
    plsc.subcore_barrier()

    def _process(row0, nr):
        pltpu.sync_copy(src_hbm.at[pl.ds(row0, nr)], sidx.at[pl.ds(0, nr)])
        pltpu.sync_copy(dst_hbm.at[pl.ds(row0, nr)], didx.at[pl.ds(0, nr)])
        waits = []
        for j in range(nr):
            waits.append(pltpu.async_copy(
                ah.at[sidx.at[j]], a_v.at[pl.ds(j * 128, 128)], gsem))
            waits.append(pltpu.async_copy(
                bh.at[didx.at[j]], b_v.at[pl.ds(j * 128, 128)], gsem))
        for w in waits:
            w.wait()

        def _ebody(i, acc):
            e = i * 4
            for u in range(4):
                a0 = a_v[e + u, 0:16]
                b0 = b_v[e + u, 0:16]
                o0 = jnp.maximum(a0 + b0, 0.0)
                o_v[e + u, 0:16] = o0
                acc = acc + o0 * o0
            return acc

        acc = lax.fori_loop(0, nr * 32, _ebody, zero16)
        s2_v[0:16] = s2_v[0:16] + acc
        for j in range(nr):
            pltpu.sync_copy(o_v.at[pl.ds(j * 128, 128)],
                            shared.at[didx.at[j]], add=True)

    base = 390 * s + jnp.minimum(s, 10)

    def _gbody(g, carry):
        _process(base + g * 8, 8)
        return carry
    lax.fori_loop(0, 48, _gbody, 0)

    rem = jnp.where(s < 10, 391, 390) - 384

    def _tbody(t, carry):
        _process(base + 384 + t, 1)
        return carry
    lax.fori_loop(0, rem, _tbody, 0)

    plsc.subcore_barrier()
    pltpu.sync_copy(shared.at[pl.ds(s * RPT, RPT)],
                    psum_out.at[c, pl.ds(s * RPT, RPT)])
    pltpu.sync_copy(s2_v, s2_out.at[c, s])


@functools.partial(
    pl.kernel,
    mesh=_mesh,
    out_type=[jax.ShapeDtypeStruct((32, NPAD), jnp.float32)],
    scratch_types=[
        pltpu.VMEM((8, 128), jnp.int32),
        pltpu.VMEM((NPAD,), jnp.float32),
    ],
    compiler_params=pltpu.CompilerParams(
        use_tc_tiling_on_sc=False, needs_layout_passes=False),
)
def _cnt_sc(dst_hbm, cnt_out, didx, hist):
    c = lax.axis_index("c")
    s = lax.axis_index("s")
    wid = s * 2 + c
    zero16 = jnp.zeros((16,), jnp.float32)
    one16 = zero16 + 1.0

    def _zh(i, carry):
        hist[pl.ds(i * 16, 16)] = zero16
        return carry
    lax.fori_loop(0, NPAD // 16, _zh, 0)

    def _process(row0, nr):
        pltpu.sync_copy(dst_hbm.at[pl.ds(row0, nr)], didx.at[pl.ds(0, nr)])
        for j in range(nr):
            for q in range(8):
                idx = didx[j, q * 16:(q + 1) * 16]
                plsc.addupdate_scatter(hist, [idx], one16)

    base = 195 * wid + jnp.minimum(wid, 10)

    def _gbody(g, carry):
        _process(base + g * 8, 8)
        return carry
    lax.fori_loop(0, 24, _gbody, 0)

    rem = jnp.where(wid < 10, 196, 195) - 192

    def _tbody(t, carry):
        _process(base + 192 + t, 1)
        return carry
    lax.fori_loop(0, rem, _tbody, 0)

    pltpu.sync_copy(hist, cnt_out.at[wid])


def _pre_body(f_ref, w1_ref, b1_ref, wt_ref, wb_ref, bc_ref, a_ref, b_ref):
    f = f_ref[...]
    w1 = w1_ref[...]
    b1 = b1_ref[...]
    acc = jnp.zeros((BN, 32), jnp.float32)
    for k in range(8):
        x0 = f[:, 2 * k:2 * k + 1]
        x1 = f[:, 2 * k + 1:2 * k + 2]
        acc = acc + jnp.maximum(x0 * w1[0:1, :] + x1 * w1[1:2, :] + b1, 0.0)
    hid = acc * 0.125
    wt = wt_ref[...]
    wb = wb_ref[...]
    bc = bc_ref[...]
    a_ref[0] = jnp.dot(hid, wt[:, 0:16], preferred_element_type=jnp.float32)
    a_ref[1] = jnp.dot(hid, wt[:, 16:32], preferred_element_type=jnp.float32)
    b_ref[0] = jnp.dot(hid, wb[:, 0:16],
                       preferred_element_type=jnp.float32) + bc[:, 0:16]
    b_ref[1] = jnp.dot(hid, wb[:, 16:32],
                       preferred_element_type=jnp.float32) + bc[:, 16:32]


def _stats1_body(p0_ref, p1_ref, s2a_ref, s2b_ref, cntp_ref,
                 sums_ref, mom_ref, cnt_ref):
    i = pl.program_id(0)
    sblk = jnp.concatenate([p0_ref[...], p1_ref[...]], axis=1)
    sums_ref[...] = sblk
    col = jnp.sum(sblk, axis=0, keepdims=True)
    ones = jnp.ones((32, 1), jnp.float32)
    cnt_ref[...] = lax.dot_general(
        cntp_ref[0], ones, (((0,), (0,)), ((), ())),
        preferred_element_type=jnp.float32)

    @pl.when(i == 0)
    def _():
        mom_ref[0:1, :] = col
        mom_ref[1:2, :] = jnp.concatenate(
            [jnp.sum(s2a_ref[...], axis=0, keepdims=True),
             jnp.sum(s2b_ref[...], axis=0, keepdims=True)], axis=1)

    @pl.when(i > 0)
    def _():
        mom_ref[0:1, :] = mom_ref[0:1, :] + col


def _stats2_body(p0_ref, p1_ref, s2a_ref, s2b_ref, sums_ref, mom_ref):
    i = pl.program_id(0)
    sblk = jnp.concatenate([p0_ref[...], p1_ref[...]], axis=1)
    sums_ref[...] = sblk
    col = jnp.sum(sblk, axis=0, keepdims=True)

    @pl.when(i == 0)
    def _():
        mom_ref[0:1, :] = col
        mom_ref[1:2, :] = jnp.concatenate(
            [jnp.sum(s2a_ref[...], axis=0, keepdims=True),
             jnp.sum(s2b_ref[...], axis=0, keepdims=True)], axis=1)

    @pl.when(i > 0)
    def _():
        mom_ref[0:1, :] = mom_ref[0:1, :] + col


def _bn_affine(mom, gamma, beta, denom_count):
    mu = mom[0:1, :] * (1.0 / denom_count)
    var = mom[1:2, :] * (1.0 / denom_count) - mu * mu
    a = gamma * lax.rsqrt(var + EPS)
    return a, beta - a * mu


def _upd1_body(sums_ref, cnt_ref, mom_ref, g_ref, be_ref, wt_ref, wb_ref,
               bc_ref, a_ref, b_ref):
    a, cadd = _bn_affine(mom_ref[...], g_ref[...], be_ref[...], E)
    cnt = cnt_ref[...]
    hid = (a * sums_ref[...] + cadd * cnt) / jnp.maximum(cnt, 1.0)
    wt = wt_ref[...]
    wb = wb_ref[...]
    bc = bc_ref[...]
    a_ref[0] = jnp.dot(hid, wt[:, 0:16], preferred_element_type=jnp.float32)
    a_ref[1] = jnp.dot(hid, wt[:, 16:32], preferred_element_type=jnp.float32)
    b_ref[0] = jnp.dot(hid, wb[:, 0:16],
                       preferred_element_type=jnp.float32) + bc[:, 0:16]
    b_ref[1] = jnp.dot(hid, wb[:, 16:32],
                       preferred_element_type=jnp.float32) + bc[:, 16:32]


def _upd2_body(sums_ref, cnt_ref, mom_ref, g_ref, be_ref, wm_ref, bm_ref,
               z_ref, zmom_ref):
    i = pl.program_id(0)
    a, cadd = _bn_affine(mom_ref[...], g_ref[...], be_ref[...], E)
    cnt = cnt_ref[...]
    hid = (a * sums_ref[...] + cadd * cnt) / jnp.maximum(cnt, 1.0)
    z = jnp.maximum(
        jnp.dot(hid, wm_ref[...], preferred_element_type=jnp.float32)
        + bm_ref[...], 0.0)
    z_ref[...] = z
    col = jnp.sum(z, axis=0, keepdims=True)
    col2 = jnp.sum(z * z, axis=0, keepdims=True)

    @pl.when(i == 0)
    def _():
        zmom_ref[0:1, :] = col
        zmom_ref[1:2, :] = col2

    @pl.when(i > 0)
    def _():
        zmom_ref[0:1, :] = zmom_ref[0:1, :] + col
        zmom_ref[1:2, :] = zmom_ref[1:2, :] + col2


def _final_body(z_ref, zmom_ref, g_ref, be_ref, wo_ref, bo_ref, out_ref):
    a, cadd = _bn_affine(zmom_ref[...], g_ref[...], be_ref[...], N)
    h2 = a * z_ref[...] + cadd
    logit = jnp.sum(h2 * wo_ref[...], axis=1, keepdims=True) + bo_ref[...]
    out_ref[...] = jax.nn.sigmoid(logit)


def _blk(shape):
    return pl.BlockSpec(shape, lambda i: (i,) + (0,) * (len(shape) - 1))


def _blk2(shape):
    return pl.BlockSpec(shape, lambda i: (0, i, 0))


def _fix(shape):
    return pl.BlockSpec(shape, lambda i: (0,) * len(shape))


def kernel(feat, edge_index, W1, b1, Wc1, bc1, gc1, bec1, Wc2, bc2, gc2, bec2,
           Wm, bm, gm, bem, Wo, bo):
    feat2 = feat.reshape(N, 16)
    src2 = edge_index[0].reshape(NROWS, 128)
    dst2 = edge_index[1].reshape(NROWS, 128)

    A1, B1 = pl.pallas_call(
        _pre_body,
        grid=(GRID,),
        in_specs=[_blk((BN, 16)), _fix((2, 32)), _fix((1, 32)),
                  _fix((32, 32)), _fix((32, 32)), _fix((1, 32))],
        out_specs=[_blk2((2, BN, 16)), _blk2((2, BN, 16))],
        out_shape=[jax.ShapeDtypeStruct((2, N, 16), jnp.float32)] * 2,
    )(feat2, W1, b1.reshape(1, 32), Wc1[:32], Wc1[32:], bc1.reshape(1, 32))

    cntp = _cnt_sc(dst2)[0]

    def edge_layer(A, B, stats_body, extra_in, extra_in_specs, extra_out,
                   extra_out_specs):
        p, s2 = _edge_sc(A, B, src2, dst2)
        return pl.pallas_call(
            stats_body,
            grid=(GRID,),
            in_specs=[_blk((BN, 16)), _blk((BN, 16)), _fix((16, 16)),
                      _fix((16, 16))] + extra_in_specs,
            out_specs=[_blk((BN, 32)), _fix((2, 32))] + extra_out_specs,
            out_shape=[jax.ShapeDtypeStruct((N, 32), jnp.float32),
                       jax.ShapeDtypeStruct((2, 32), jnp.float32)] + extra_out,
        )(p[0, :N], p[1, :N], s2[0], s2[1], *extra_in)

    cntp3 = cntp[:, :N].reshape(32, GRID, BN).transpose(1, 0, 2)
    sums1, mom1, cntc = edge_layer(
        A1, B1, _stats1_body,
        [cntp3], [pl.BlockSpec((1, 32, BN), lambda i: (i, 0, 0))],
        [jax.ShapeDtypeStruct((N, 1), jnp.float32)], [_blk((BN, 1))])

    A2, B2 = pl.pallas_call(
        _upd1_body,
        grid=(GRID,),
        in_specs=[_blk((BN, 32)), _blk((BN, 1)), _fix((2, 32)),
                  _fix((1, 32)), _fix((1, 32)), _fix((32, 32)),
                  _fix((32, 32)), _fix((1, 32))],
        out_specs=[_blk2((2, BN, 16)), _blk2((2, BN, 16))],
        out_shape=[jax.ShapeDtypeStruct((2, N, 16), jnp.float32)] * 2,
    )(sums1, cntc, mom1, gc1.reshape(1, 32), bec1.reshape(1, 32),
      Wc2[:32], Wc2[32:], bc2.reshape(1, 32))

    sums2, mom2 = edge_layer(A2, B2, _stats2_body, [], [], [], [])

    z, zmom = pl.pallas_call(
        _upd2_body,
        grid=(GRID,),
        in_specs=[_blk((BN, 32)), _blk((BN, 1)), _fix((2, 32)),
                  _fix((1, 32)), _fix((1, 32)), _fix((32, 16)),
                  _fix((1, 16))],
        out_specs=[_blk((BN, 16)), _fix((2, 16))],
        out_shape=[jax.ShapeDtypeStruct((N, 16), jnp.float32),
                   jax.ShapeDtypeStruct((2, 16), jnp.float32)],
    )(sums2, cntc, mom2, gc2.reshape(1, 32), bec2.reshape(1, 32),
      Wm, bm.reshape(1, 16))

    out = pl.pallas_call(
        _final_body,
        grid=(GRID,),
        in_specs=[_blk((BN, 16)), _fix((2, 16)), _fix((1, 16)),
                  _fix((1, 16)), _fix((1, 16)), _fix((1, 1))],
        out_specs=_blk((BN, 1)),
        out_shape=jax.ShapeDtypeStruct((N, 1), jnp.float32),
    )(z, zmom, gm.reshape(1, 16), bem.reshape(1, 16),
      Wo.reshape(1, 16), bo.reshape(1, 1))

    return out


# speedup vs baseline: 9.6734x; 1.2082x over previous
import functools

import jax
import jax.numpy as jnp
from jax import lax
from jax.experimental import pallas as pl
from jax.experimental.pallas import tpu as pltpu
from jax.experimental.pallas import tpu_sc as plsc

N = 50000
E = 800000
EPS = 1e-5
NROWS = E // 128
NPAD = 50176
RPT = NPAD // 16
ZR = RPT // 8
BN = 2000
GRID = N // BN

_mesh = plsc.VectorSubcoreMesh(core_axis_name="c", subcore_axis_name="s")
_sc_params = pltpu.CompilerParams(use_tc_tiling_on_sc=False)


@functools.partial(
    pl.kernel,
    mesh=_mesh,
    out_type=[
        jax.ShapeDtypeStruct((2, NPAD, 16), jnp.float32),
        jax.ShapeDtypeStruct((2, 16, 16), jnp.float32),
    ],
    scratch_types=[
        pltpu.VMEM((8, 128), jnp.int32),
        pltpu.VMEM((8, 128), jnp.int32),
        pltpu.VMEM((1024, 16), jnp.float32),
        pltpu.VMEM((1024, 16), jnp.float32),
        pltpu.VMEM((1024, 16), jnp.float32),
        pltpu.VMEM((16,), jnp.float32),
        pltpu.VMEM((ZR, 16), jnp.float32),
        pltpu.VMEM_SHARED((NPAD, 16), jnp.float32),
        pltpu.SemaphoreType.DMA,
    ],
    compiler_params=_sc_params,
)
def _edge_sc(a_hbm, b_hbm, src_hbm, dst_hbm, psum_out, s2_out,
             sidx, didx, a_v, b_v, o_v, s2_v, zbuf, shared, gsem):
    c = lax.axis_index("c")
    s = lax.axis_index("s")
    zero16 = jnp.zeros((16,), jnp.float32)
    ah = a_hbm.at[c]
    bh = b_hbm.at[c]

    def _zb(r, carry):
        zbuf[r, 0:16] = zero16
        return carry
    lax.fori_loop(0, ZR, _zb, 0)
    for t in range(8):
        pltpu.sync_copy(zbuf, shared.at[pl.ds(s * RPT + t * ZR, ZR)])
    s2_v[0:16] = zero16
    plsc.subcore_barrier()

    def _process(row0, nr):
        pltpu.sync_copy(src_hbm.at[pl.ds(row0, nr)], sidx.at[pl.ds(0, nr)])
        pltpu.sync_copy(dst_hbm.at[pl.ds(row0, nr)], didx.at[pl.ds(0, nr)])
        waits = []
        for j in range(nr):
            waits.append(pltpu.async_copy(
                ah.at[sidx.at[j]], a_v.at[pl.ds(j * 128, 128)], gsem))
            waits.append(pltpu.async_copy(
                bh.at[didx.at[j]], b_v.at[pl.ds(j * 128, 128)], gsem))
        for w in waits:
            w.wait()

        def _ebody(i, acc):
            e = i * 4
            for u in range(4):
                a0 = a_v[e + u, 0:16]
                b0 = b_v[e + u, 0:16]
                o0 = jnp.maximum(a0 + b0, 0.0)
                o_v[e + u, 0:16] = o0
                acc = acc + o0 * o0
            return acc

        acc = lax.fori_loop(0, nr * 32, _ebody, zero16)
        s2_v[0:16] = s2_v[0:16] + acc
        for j in range(nr):
            pltpu.sync_copy(o_v.at[pl.ds(j * 128, 128)],
                            shared.at[didx.at[j]], add=True)

    base = 390 * s + jnp.minimum(s, 10)

    def _gbody(g, carry):
        _process(base + g * 8, 8)
        return carry
    lax.fori_loop(0, 48, _gbody, 0)

    rem = jnp.where(s < 10, 391, 390) - 384

    def _tbody(t, carry):
        _process(base + 384 + t, 1)
        return carry
    lax.fori_loop(0, rem, _tbody, 0)

    plsc.subcore_barrier()
    pltpu.sync_copy(shared.at[pl.ds(s * RPT, RPT)],
                    psum_out.at[c, pl.ds(s * RPT, RPT)])
    pltpu.sync_copy(s2_v, s2_out.at[c, s])


@functools.partial(
    pl.kernel,
    mesh=_mesh,
    out_type=[jax.ShapeDtypeStruct((32, NPAD), jnp.float32)],
    scratch_types=[
        pltpu.VMEM((8, 128), jnp.int32),
        pltpu.VMEM((NPAD,), jnp.float32),
    ],
    compiler_params=pltpu.CompilerParams(
        use_tc_tiling_on_sc=False, needs_layout_passes=False),
)
def _cnt_sc(dst_hbm, cnt_out, didx, hist):
    c = lax.axis_index("c")
    s = lax.axis_index("s")
    wid = s * 2 + c
    zero16 = jnp.zeros((16,), jnp.float32)
    one16 = zero16 + 1.0

    def _zh(i, carry):
        hist[pl.ds(i * 16, 16)] = zero16
        return carry
    lax.fori_loop(0, NPAD // 16, _zh, 0)

    def _process(row0, nr):
        pltpu.sync_copy(dst_hbm.at[pl.ds(row0, nr)], didx.at[pl.ds(0, nr)])
        for j in range(nr):
            for q in range(8):
                idx = didx[j, q * 16:(q + 1) * 16]
                plsc.addupdate_scatter(hist, [idx], one16)

    base = 195 * wid + jnp.minimum(wid, 10)

    def _gbody(g, carry):
        _process(base + g * 8, 8)
        return carry
    lax.fori_loop(0, 24, _gbody, 0)

    rem = jnp.where(wid < 10, 196, 195) - 192

    def _tbody(t, carry):
        _process(base + 192 + t, 1)
        return carry
    lax.fori_loop(0, rem, _tbody, 0)

    pltpu.sync_copy(hist, cnt_out.at[wid])


def _pre_body(f_ref, w1e_ref, b1e_ref, s_ref, wt_ref, wb_ref, bc_ref,
              a_ref, b_ref):
    y = jnp.maximum(
        jnp.dot(f_ref[...], w1e_ref[...], preferred_element_type=jnp.float32)
        + b1e_ref[...], 0.0)
    hid = jnp.dot(y, s_ref[...], preferred_element_type=jnp.float32)
    wt = wt_ref[...]
    wb = wb_ref[...]
    bc = bc_ref[...]
    a_ref[0] = jnp.dot(hid, wt[:, 0:16], preferred_element_type=jnp.float32)
    a_ref[1] = jnp.dot(hid, wt[:, 16:32], preferred_element_type=jnp.float32)
    b_ref[0] = jnp.dot(hid, wb[:, 0:16],
                       preferred_element_type=jnp.float32) + bc[:, 0:16]
    b_ref[1] = jnp.dot(hid, wb[:, 16:32],
                       preferred_element_type=jnp.float32) + bc[:, 16:32]


def _stats_body(pa_ref, pb_ref, s2_ref, sums_ref, mom_ref):
    i = pl.program_id(0)
    sblk = jnp.concatenate([pa_ref[0], pb_ref[0]], axis=1)
    sums_ref[...] = sblk
    col = jnp.sum(sblk, axis=0, keepdims=True)

    @pl.when(i == 0)
    def _():
        mom_ref[0:1, :] = col
        mom_ref[1:2, :] = jnp.concatenate(
            [jnp.sum(s2_ref[0], axis=0, keepdims=True),
             jnp.sum(s2_ref[1], axis=0, keepdims=True)], axis=1)

    @pl.when(i > 0)
    def _():
        mom_ref[0:1, :] = mom_ref[0:1, :] + col


def _cntred_body(cntp_ref, cnt_ref):
    ones = jnp.ones((32, 1), jnp.float32)
    cnt_ref[...] = lax.dot_general(
        cntp_ref[0], ones, (((0,), (0,)), ((), ())),
        preferred_element_type=jnp.float32)


def _bn_affine(mom, gamma, beta, denom_count):
    mu = mom[0:1, :] * (1.0 / denom_count)
    var = mom[1:2, :] * (1.0 / denom_count) - mu * mu
    a = gamma * lax.rsqrt(var + EPS)
    return a, beta - a * mu


def _upd1_body(sums_ref, cnt_ref, mom_ref, g_ref, be_ref, wt_ref, wb_ref,
               bc_ref, a_ref, b_ref):
    a, cadd = _bn_affine(mom_ref[...], g_ref[...], be_ref[...], E)
    cnt = cnt_ref[...]
    hid = (a * sums_ref[...] + cadd * cnt) / jnp.maximum(cnt, 1.0)
    wt = wt_ref[...]
    wb = wb_ref[...]
    bc = bc_ref[...]
    a_ref[0] = jnp.dot(hid, wt[:, 0:16], preferred_element_type=jnp.float32)
    a_ref[1] = jnp.dot(hid, wt[:, 16:32], preferred_element_type=jnp.float32)
    b_ref[0] = jnp.dot(hid, wb[:, 0:16],
                       preferred_element_type=jnp.float32) + bc[:, 0:16]
    b_ref[1] = jnp.dot(hid, wb[:, 16:32],
                       preferred_element_type=jnp.float32) + bc[:, 16:32]


def _upd2_body(sums_ref, cnt_ref, mom_ref, g_ref, be_ref, wm_ref, bm_ref,
               z_ref, zmom_ref):
    i = pl.program_id(0)
    a, cadd = _bn_affine(mom_ref[...], g_ref[...], be_ref[...], E)
    cnt = cnt_ref[...]
    hid = (a * sums_ref[...] + cadd * cnt) / jnp.maximum(cnt, 1.0)
    z = jnp.maximum(
        jnp.dot(hid, wm_ref[...], preferred_element_type=jnp.float32)
        + bm_ref[...], 0.0)
    z_ref[...] = z
    col = jnp.sum(z, axis=0, keepdims=True)
    col2 = jnp.sum(z * z, axis=0, keepdims=True)

    @pl.when(i == 0)
    def _():
        zmom_ref[0:1, :] = col
        zmom_ref[1:2, :] = col2

    @pl.when(i > 0)
    def _():
        zmom_ref[0:1, :] = zmom_ref[0:1, :] + col
        zmom_ref[1:2, :] = zmom_ref[1:2, :] + col2


def _final_body(z_ref, zmom_ref, g_ref, be_ref, wo_ref, bo_ref, out_ref):
    a, cadd = _bn_affine(zmom_ref[...], g_ref[...], be_ref[...], N)
    h2 = a * z_ref[...] + cadd
    logit = jnp.sum(h2 * wo_ref[...], axis=1, keepdims=True) + bo_ref[...]
    out_ref[...] = jax.nn.sigmoid(logit)


def _blk(shape):
    return pl.BlockSpec(shape, lambda i: (i,) + (0,) * (len(shape) - 1))


def _blk2(shape):
    return pl.BlockSpec(shape, lambda i: (0, i, 0))


def _fix(shape):
    return pl.BlockSpec(shape, lambda i: (0,) * len(shape))


def kernel(feat, edge_index, W1, b1, Wc1, bc1, gc1, bec1, Wc2, bc2, gc2, bec2,
           Wm, bm, gm, bem, Wo, bo):
    feat2 = feat.reshape(N, 16)
    src2 = edge_index[0].reshape(NROWS, 128)
    dst2 = edge_index[1].reshape(NROWS, 128)
    W1e = jnp.kron(jnp.eye(8, dtype=jnp.float32), W1)
    b1e = jnp.tile(b1, 8).reshape(1, 256)
    S8 = jnp.tile(jnp.eye(32, dtype=jnp.float32) * 0.125, (8, 1))

    A1, B1 = pl.pallas_call(
        _pre_body,
        grid=(GRID,),
        in_specs=[_blk((BN, 16)), _fix((16, 256)), _fix((1, 256)),
                  _fix((256, 32)), _fix((32, 32)), _fix((32, 32)),
                  _fix((1, 32))],
        out_specs=[_blk2((2, BN, 16)), _blk2((2, BN, 16))],
        out_shape=[jax.ShapeDtypeStruct((2, N, 16), jnp.float32)] * 2,
    )(feat2, W1e, b1e, S8, Wc1[:32], Wc1[32:], bc1.reshape(1, 32))

    cntp = _cnt_sc(dst2)[0]
    cntp3 = cntp[:, :N].reshape(32, GRID, BN).transpose(1, 0, 2)
    cntc = pl.pallas_call(
        _cntred_body,
        grid=(GRID,),
        in_specs=[pl.BlockSpec((1, 32, BN), lambda i: (i, 0, 0))],
        out_specs=_blk((BN, 1)),
        out_shape=jax.ShapeDtypeStruct((N, 1), jnp.float32),
    )(cntp3)

    def edge_layer(A, B):
        p, s2 = _edge_sc(A, B, src2, dst2)
        return pl.pallas_call(
            _stats_body,
            grid=(GRID,),
            in_specs=[_blk2((1, BN, 16)),
                      pl.BlockSpec((1, BN, 16), lambda i: (1, i, 0)),
                      _fix((2, 16, 16))],
            out_specs=[_blk((BN, 32)), _fix((2, 32))],
            out_shape=[jax.ShapeDtypeStruct((N, 32), jnp.float32),
                       jax.ShapeDtypeStruct((2, 32), jnp.float32)],
        )(p, p, s2)

    sums1, mom1 = edge_layer(A1, B1)

    A2, B2 = pl.pallas_call(
        _upd1_body,
        grid=(GRID,),
        in_specs=[_blk((BN, 32)), _blk((BN, 1)), _fix((2, 32)),
                  _fix((1, 32)), _fix((1, 32)), _fix((32, 32)),
                  _fix((32, 32)), _fix((1, 32))],
        out_specs=[_blk2((2, BN, 16)), _blk2((2, BN, 16))],
        out_shape=[jax.ShapeDtypeStruct((2, N, 16), jnp.float32)] * 2,
    )(sums1, cntc, mom1, gc1.reshape(1, 32), bec1.reshape(1, 32),
      Wc2[:32], Wc2[32:], bc2.reshape(1, 32))

    sums2, mom2 = edge_layer(A2, B2)

    z, zmom = pl.pallas_call(
        _upd2_body,
        grid=(GRID,),
        in_specs=[_blk((BN, 32)), _blk((BN, 1)), _fix((2, 32)),
                  _fix((1, 32)), _fix((1, 32)), _fix((32, 16)),
                  _fix((1, 16))],
        out_specs=[_blk((BN, 16)), _fix((2, 16))],
        out_shape=[jax.ShapeDtypeStruct((N, 16), jnp.float32),
                   jax.ShapeDtypeStruct((2, 16), jnp.float32)],
    )(sums2, cntc, mom2, gc2.reshape(1, 32), bec2.reshape(1, 32),
      Wm, bm.reshape(1, 16))

    out = pl.pallas_call(
        _final_body,
        grid=(GRID,),
        in_specs=[_blk((BN, 16)), _fix((2, 16)), _fix((1, 16)),
                  _fix((1, 16)), _fix((1, 16)), _fix((1, 1))],
        out_specs=_blk((BN, 1)),
        out_shape=jax.ShapeDtypeStruct((N, 1), jnp.float32),
    )(z, zmom, gm.reshape(1, 16), bem.reshape(1, 16),
      Wo.reshape(1, 16), bo.reshape(1, 1))

    return out


# speedup vs baseline: 12.2213x; 1.2634x over previous
import functools

import jax
import jax.numpy as jnp
from jax import lax
from jax.experimental import pallas as pl
from jax.experimental.pallas import tpu as pltpu
from jax.experimental.pallas import tpu_sc as plsc

N = 50000
E = 800000
EPS = 1e-5
NROWS = E // 128
NPAD = 50176
RPT = NPAD // 16
ZR = RPT // 8
BN = 2000
GRID = N // BN

_mesh = plsc.VectorSubcoreMesh(core_axis_name="c", subcore_axis_name="s")
_sc_params = pltpu.CompilerParams(use_tc_tiling_on_sc=False)


@functools.partial(
    pl.kernel,
    mesh=_mesh,
    out_type=[
        jax.ShapeDtypeStruct((2, NPAD, 16), jnp.float32),
        jax.ShapeDtypeStruct((2, 16, 16), jnp.float32),
    ],
    scratch_types=[
        pltpu.VMEM((8, 128), jnp.int32),
        pltpu.VMEM((8, 128), jnp.int32),
        pltpu.VMEM((8, 128), jnp.int32),
        pltpu.VMEM((8, 128), jnp.int32),
        pltpu.VMEM((1024, 16), jnp.float32),
        pltpu.VMEM((1024, 16), jnp.float32),
        pltpu.VMEM((1024, 16), jnp.float32),
        pltpu.VMEM((1024, 16), jnp.float32),
        pltpu.VMEM((16,), jnp.float32),
        pltpu.VMEM_SHARED((NPAD, 16), jnp.float32),
        pltpu.SemaphoreType.DMA,
        pltpu.SemaphoreType.DMA,
    ],
    compiler_params=_sc_params,
)
def _edge_sc(a_hbm, b_hbm, src_hbm, dst_hbm, psum_out, s2_out,
             si0, di0, si1, di1, a0, b0, a1, b1,
             s2_v, shared, gsem0, gsem1):
    c = lax.axis_index("c")
    s = lax.axis_index("s")
    zero16 = jnp.zeros((16,), jnp.float32)
    ah = a_hbm.at[c]
    bh = b_hbm.at[c]

    def _zb(r, carry):
        a0[r, 0:16] = zero16
        return carry
    lax.fori_loop(0, ZR, _zb, 0)
    for t in range(8):
        pltpu.sync_copy(a0.at[pl.ds(0, ZR)],
                        shared.at[pl.ds(s * RPT + t * ZR, ZR)])
    s2_v[0:16] = zero16

    base = 390 * s + jnp.minimum(s, 10)

    def idx_load(g, si, di):
        r = base + g * 8
        pltpu.sync_copy(src_hbm.at[pl.ds(r, 8)], si)
        pltpu.sync_copy(dst_hbm.at[pl.ds(r, 8)], di)

    def fire(si, di, av, bv, sem):
        for j in range(8):
            pltpu.async_copy(ah.at[si.at[j]], av.at[pl.ds(j * 128, 128)], sem)
            pltpu.async_copy(bh.at[di.at[j]], bv.at[pl.ds(j * 128, 128)], sem)

    def wait(si, di, av, bv, sem):
        for j in range(8):
            pltpu.make_async_copy(
                ah.at[si.at[j]], av.at[pl.ds(j * 128, 128)], sem).wait()
            pltpu.make_async_copy(
                bh.at[di.at[j]], bv.at[pl.ds(j * 128, 128)], sem).wait()

    def scat(ov, di):
        for j in range(8):
            pltpu.sync_copy(ov.at[pl.ds(j * 128, 128)],
                            shared.at[di.at[j]], add=True)

    def compute(av, bv, n128):
        def _ebody(i, acc):
            e = i * 4
            for u in range(4):
                x = jnp.maximum(av[e + u, 0:16] + bv[e + u, 0:16], 0.0)
                av[e + u, 0:16] = x
                acc = acc + x * x
            return acc
        acc = lax.fori_loop(0, n128 * 32, _ebody, zero16)
        s2_v[0:16] = s2_v[0:16] + acc

    idx_load(0, si0, di0)
    fire(si0, di0, a0, b0, gsem0)
    idx_load(1, si1, di1)
    fire(si1, di1, a1, b1, gsem1)
    plsc.subcore_barrier()

    def body(i, carry):
        g = 2 * i
        wait(si0, di0, a0, b0, gsem0)
        compute(a0, b0, 8)
        scat(a0, di0)
        idx_load(g + 2, si0, di0)
        fire(si0, di0, a0, b0, gsem0)

        wait(si1, di1, a1, b1, gsem1)
        compute(a1, b1, 8)
        scat(a1, di1)
        idx_load(g + 3, si1, di1)
        fire(si1, di1, a1, b1, gsem1)
        return carry

    lax.fori_loop(0, 23, body, 0)

    wait(si0, di0, a0, b0, gsem0)
    compute(a0, b0, 8)
    scat(a0, di0)
    wait(si1, di1, a1, b1, gsem1)
    compute(a1, b1, 8)
    scat(a1, di1)

    rem = jnp.where(s < 10, 391, 390) - 384

    def _tbody(t, carry):
        r = base + 384 + t
        sit = si0.at[0]
        dit = di0.at[0]
        avt = a0.at[pl.ds(0, 128)]
        bvt = b0.at[pl.ds(0, 128)]
        pltpu.sync_copy(src_hbm.at[pl.ds(r, 1)], si0.at[pl.ds(0, 1)])
        pltpu.sync_copy(dst_hbm.at[pl.ds(r, 1)], di0.at[pl.ds(0, 1)])
        pltpu.async_copy(ah.at[sit], avt, gsem0)
        pltpu.async_copy(bh.at[dit], bvt, gsem0)
        pltpu.make_async_copy(ah.at[sit], avt, gsem0).wait()
        pltpu.make_async_copy(bh.at[dit], bvt, gsem0).wait()
        compute(a0, b0, 1)
        pltpu.sync_copy(a0.at[pl.ds(0, 128)], shared.at[dit], add=True)
        return carry

    lax.fori_loop(0, rem, _tbody, 0)

    plsc.subcore_barrier()
    pltpu.sync_copy(shared.at[pl.ds(s * RPT, RPT)],
                    psum_out.at[c, pl.ds(s * RPT, RPT)])
    pltpu.sync_copy(s2_v, s2_out.at[c, s])


@functools.partial(
    pl.kernel,
    mesh=_mesh,
    out_type=[jax.ShapeDtypeStruct((32, NPAD), jnp.float32)],
    scratch_types=[
        pltpu.VMEM((8, 128), jnp.int32),
        pltpu.VMEM((NPAD,), jnp.float32),
    ],
    compiler_params=pltpu.CompilerParams(
        use_tc_tiling_on_sc=False, needs_layout_passes=False),
)
def _cnt_sc(dst_hbm, cnt_out, didx, hist):
    c = lax.axis_index("c")
    s = lax.axis_index("s")
    wid = s * 2 + c
    zero16 = jnp.zeros((16,), jnp.float32)
    one16 = zero16 + 1.0

    def _zh(i, carry):
        hist[pl.ds(i * 16, 16)] = zero16
        return carry
    lax.fori_loop(0, NPAD // 16, _zh, 0)

    def _process(row0, nr):
        pltpu.sync_copy(dst_hbm.at[pl.ds(row0, nr)], didx.at[pl.ds(0, nr)])
        for j in range(nr):
            for q in range(8):
                idx = didx[j, q * 16:(q + 1) * 16]
                plsc.addupdate_scatter(hist, [idx], one16)

    base = 195 * wid + jnp.minimum(wid, 10)

    def _gbody(g, carry):
        _process(base + g * 8, 8)
        return carry
    lax.fori_loop(0, 24, _gbody, 0)

    rem = jnp.where(wid < 10, 196, 195) - 192

    def _tbody(t, carry):
        _process(base + 192 + t, 1)
        return carry
    lax.fori_loop(0, rem, _tbody, 0)

    pltpu.sync_copy(hist, cnt_out.at[wid])


def _pre_body(f_ref, w1e_ref, b1e_ref, s_ref, wt_ref, wb_ref, bc_ref,
              a_ref, b_ref):
    y = jnp.maximum(
        jnp.dot(f_ref[...], w1e_ref[...], preferred_element_type=jnp.float32)
        + b1e_ref[...], 0.0)
    hid = jnp.dot(y, s_ref[...], preferred_element_type=jnp.float32)
    wt = wt_ref[...]
    wb = wb_ref[...]
    bc = bc_ref[...]
    a_ref[0] = jnp.dot(hid, wt[:, 0:16], preferred_element_type=jnp.float32)
    a_ref[1] = jnp.dot(hid, wt[:, 16:32], preferred_element_type=jnp.float32)
    b_ref[0] = jnp.dot(hid, wb[:, 0:16],
                       preferred_element_type=jnp.float32) + bc[:, 0:16]
    b_ref[1] = jnp.dot(hid, wb[:, 16:32],
                       preferred_element_type=jnp.float32) + bc[:, 16:32]


def _stats_body(pa_ref, pb_ref, s2_ref, sums_ref, mom_ref):
    i = pl.program_id(0)
    sblk = jnp.concatenate([pa_ref[0], pb_ref[0]], axis=1)
    sums_ref[...] = sblk
    col = jnp.sum(sblk, axis=0, keepdims=True)

    @pl.when(i == 0)
    def _():
        mom_ref[0:1, :] = col
        mom_ref[1:2, :] = jnp.concatenate(
            [jnp.sum(s2_ref[0], axis=0, keepdims=True),
             jnp.sum(s2_ref[1], axis=0, keepdims=True)], axis=1)

    @pl.when(i > 0)
    def _():
        mom_ref[0:1, :] = mom_ref[0:1, :] + col


def _cntred_body(cntp_ref, cnt_ref):
    ones = jnp.ones((32, 1), jnp.float32)
    cnt_ref[...] = lax.dot_general(
        cntp_ref[0], ones, (((0,), (0,)), ((), ())),
        preferred_element_type=jnp.float32)


def _bn_affine(mom, gamma, beta, denom_count):
    mu = mom[0:1, :] * (1.0 / denom_count)
    var = mom[1:2, :] * (1.0 / denom_count) - mu * mu
    a = gamma * lax.rsqrt(var + EPS)
    return a, beta - a * mu


def _upd1_body(sums_ref, cnt_ref, mom_ref, g_ref, be_ref, wt_ref, wb_ref,
               bc_ref, a_ref, b_ref):
    a, cadd = _bn_affine(mom_ref[...], g_ref[...], be_ref[...], E)
    cnt = cnt_ref[...]
    hid = (a * sums_ref[...] + cadd * cnt) / jnp.maximum(cnt, 1.0)
    wt = wt_ref[...]
    wb = wb_ref[...]
    bc = bc_ref[...]
    a_ref[0] = jnp.dot(hid, wt[:, 0:16], preferred_element_type=jnp.float32)
    a_ref[1] = jnp.dot(hid, wt[:, 16:32], preferred_element_type=jnp.float32)
    b_ref[0] = jnp.dot(hid, wb[:, 0:16],
                       preferred_element_type=jnp.float32) + bc[:, 0:16]
    b_ref[1] = jnp.dot(hid, wb[:, 16:32],
                       preferred_element_type=jnp.float32) + bc[:, 16:32]


def _upd2_body(sums_ref, cnt_ref, mom_ref, g_ref, be_ref, wm_ref, bm_ref,
               z_ref, zmom_ref):
    i = pl.program_id(0)
    a, cadd = _bn_affine(mom_ref[...], g_ref[...], be_ref[...], E)
    cnt = cnt_ref[...]
    hid = (a * sums_ref[...] + cadd * cnt) / jnp.maximum(cnt, 1.0)
    z = jnp.maximum(
        jnp.dot(hid, wm_ref[...], preferred_element_type=jnp.float32)
        + bm_ref[...], 0.0)
    z_ref[...] = z
    col = jnp.sum(z, axis=0, keepdims=True)
    col2 = jnp.sum(z * z, axis=0, keepdims=True)

    @pl.when(i == 0)
    def _():
        zmom_ref[0:1, :] = col
        zmom_ref[1:2, :] = col2

    @pl.when(i > 0)
    def _():
        zmom_ref[0:1, :] = zmom_ref[0:1, :] + col
        zmom_ref[1:2, :] = zmom_ref[1:2, :] + col2


def _final_body(z_ref, zmom_ref, g_ref, be_ref, wo_ref, bo_ref, out_ref):
    a, cadd = _bn_affine(zmom_ref[...], g_ref[...], be_ref[...], N)
    h2 = a * z_ref[...] + cadd
    logit = jnp.sum(h2 * wo_ref[...], axis=1, keepdims=True) + bo_ref[...]
    out_ref[...] = jax.nn.sigmoid(logit)


def _blk(shape):
    return pl.BlockSpec(shape, lambda i: (i,) + (0,) * (len(shape) - 1))


def _blk2(shape):
    return pl.BlockSpec(shape, lambda i: (0, i, 0))


def _fix(shape):
    return pl.BlockSpec(shape, lambda i: (0,) * len(shape))


def kernel(feat, edge_index, W1, b1, Wc1, bc1, gc1, bec1, Wc2, bc2, gc2, bec2,
           Wm, bm, gm, bem, Wo, bo):
    feat2 = feat.reshape(N, 16)
    src2 = edge_index[0].reshape(NROWS, 128)
    dst2 = edge_index[1].reshape(NROWS, 128)
    W1e = jnp.kron(jnp.eye(8, dtype=jnp.float32), W1)
    b1e = jnp.tile(b1, 8).reshape(1, 256)
    S8 = jnp.tile(jnp.eye(32, dtype=jnp.float32) * 0.125, (8, 1))

    A1, B1 = pl.pallas_call(
        _pre_body,
        grid=(GRID,),
        in_specs=[_blk((BN, 16)), _fix((16, 256)), _fix((1, 256)),
                  _fix((256, 32)), _fix((32, 32)), _fix((32, 32)),
                  _fix((1, 32))],
        out_specs=[_blk2((2, BN, 16)), _blk2((2, BN, 16))],
        out_shape=[jax.ShapeDtypeStruct((2, N, 16), jnp.float32)] * 2,
    )(feat2, W1e, b1e, S8, Wc1[:32], Wc1[32:], bc1.reshape(1, 32))

    cntp = _cnt_sc(dst2)[0]
    cntp3 = cntp[:, :N].reshape(32, GRID, BN).transpose(1, 0, 2)
    cntc = pl.pallas_call(
        _cntred_body,
        grid=(GRID,),
        in_specs=[pl.BlockSpec((1, 32, BN), lambda i: (i, 0, 0))],
        out_specs=_blk((BN, 1)),
        out_shape=jax.ShapeDtypeStruct((N, 1), jnp.float32),
    )(cntp3)

    def edge_layer(A, B):
        p, s2 = _edge_sc(A, B, src2, dst2)
        return pl.pallas_call(
            _stats_body,
            grid=(GRID,),
            in_specs=[_blk2((1, BN, 16)),
                      pl.BlockSpec((1, BN, 16), lambda i: (1, i, 0)),
                      _fix((2, 16, 16))],
            out_specs=[_blk((BN, 32)), _fix((2, 32))],
            out_shape=[jax.ShapeDtypeStruct((N, 32), jnp.float32),
                       jax.ShapeDtypeStruct((2, 32), jnp.float32)],
        )(p, p, s2)

    sums1, mom1 = edge_layer(A1, B1)

    A2, B2 = pl.pallas_call(
        _upd1_body,
        grid=(GRID,),
        in_specs=[_blk((BN, 32)), _blk((BN, 1)), _fix((2, 32)),
                  _fix((1, 32)), _fix((1, 32)), _fix((32, 32)),
                  _fix((32, 32)), _fix((1, 32))],
        out_specs=[_blk2((2, BN, 16)), _blk2((2, BN, 16))],
        out_shape=[jax.ShapeDtypeStruct((2, N, 16), jnp.float32)] * 2,
    )(sums1, cntc, mom1, gc1.reshape(1, 32), bec1.reshape(1, 32),
      Wc2[:32], Wc2[32:], bc2.reshape(1, 32))

    sums2, mom2 = edge_layer(A2, B2)

    z, zmom = pl.pallas_call(
        _upd2_body,
        grid=(GRID,),
        in_specs=[_blk((BN, 32)), _blk((BN, 1)), _fix((2, 32)),
                  _fix((1, 32)), _fix((1, 32)), _fix((32, 16)),
                  _fix((1, 16))],
        out_specs=[_blk((BN, 16)), _fix((2, 16))],
        out_shape=[jax.ShapeDtypeStruct((N, 16), jnp.float32),
                   jax.ShapeDtypeStruct((2, 16), jnp.float32)],
    )(sums2, cntc, mom2, gc2.reshape(1, 32), bec2.reshape(1, 32),
      Wm, bm.reshape(1, 16))

    out = pl.pallas_call(
        _final_body,
        grid=(GRID,),
        in_specs=[_blk((BN, 16)), _fix((2, 16)), _fix((1, 16)),
                  _fix((1, 16)), _fix((1, 16)), _fix((1, 1))],
        out_specs=_blk((BN, 1)),
        out_shape=jax.ShapeDtypeStruct((N, 1), jnp.float32),
    )(z, zmom, gm.reshape(1, 16), bem.reshape(1, 16),
      Wo.reshape(1, 16), bo.reshape(1, 1))

    return out


# speedup vs baseline: 12.9081x; 1.0562x over previous
import functools

import jax
import jax.numpy as jnp
from jax import lax
from jax.experimental import pallas as pl
from jax.experimental.pallas import tpu as pltpu
from jax.experimental.pallas import tpu_sc as plsc

N = 50000
E = 800000
EPS = 1e-5
NROWS = E // 128
NPAD = 50176
RPT = NPAD // 16
ZR = RPT // 8
BN = 5000
GRID = N // BN

_mesh = plsc.VectorSubcoreMesh(core_axis_name="c", subcore_axis_name="s")
_sc_params = pltpu.CompilerParams(use_tc_tiling_on_sc=False)


@functools.partial(
    pl.kernel,
    mesh=_mesh,
    out_type=[
        jax.ShapeDtypeStruct((2, NPAD, 16), jnp.float32),
        jax.ShapeDtypeStruct((2, 16, 16), jnp.float32),
    ],
    scratch_types=[
        pltpu.VMEM((8, 128), jnp.int32),
        pltpu.VMEM((8, 128), jnp.int32),
        pltpu.VMEM((8, 128), jnp.int32),
        pltpu.VMEM((8, 128), jnp.int32),
        pltpu.VMEM((1024, 16), jnp.float32),
        pltpu.VMEM((1024, 16), jnp.float32),
        pltpu.VMEM((1024, 16), jnp.float32),
        pltpu.VMEM((1024, 16), jnp.float32),
        pltpu.VMEM((16,), jnp.float32),
        pltpu.VMEM_SHARED((NPAD, 16), jnp.float32),
        pltpu.SemaphoreType.DMA,
        pltpu.SemaphoreType.DMA,
        pltpu.SemaphoreType.DMA,
        pltpu.SemaphoreType.DMA,
    ],
    compiler_params=_sc_params,
)
def _edge_sc(a_hbm, b_hbm, src_hbm, dst_hbm, psum_out, s2_out,
             si0, di0, si1, di1, a0, b0, a1, b1,
             s2_v, shared, gsem0, gsem1, ssem0, ssem1):
    c = lax.axis_index("c")
    s = lax.axis_index("s")
    zero16 = jnp.zeros((16,), jnp.float32)
    ah = a_hbm.at[c]
    bh = b_hbm.at[c]

    def _zb(r, carry):
        a0[r, 0:16] = zero16
        return carry
    lax.fori_loop(0, ZR, _zb, 0)
    for t in range(8):
        pltpu.sync_copy(a0.at[pl.ds(0, ZR)],
                        shared.at[pl.ds(s * RPT + t * ZR, ZR)])
    s2_v[0:16] = zero16

    base = 390 * s + jnp.minimum(s, 10)

    def idx_load(g, si, di):
        r = base + g * 8
        pltpu.sync_copy(src_hbm.at[pl.ds(r, 8)], si)
        pltpu.sync_copy(dst_hbm.at[pl.ds(r, 8)], di)

    def fire(si, di, av, bv, sem):
        for j in range(8):
            pltpu.async_copy(ah.at[si.at[j]], av.at[pl.ds(j * 128, 128)], sem)
            pltpu.async_copy(bh.at[di.at[j]], bv.at[pl.ds(j * 128, 128)], sem)

    def wait(si, di, av, bv, sem):
        for j in range(8):
            pltpu.make_async_copy(
                ah.at[si.at[j]], av.at[pl.ds(j * 128, 128)], sem).wait()
            pltpu.make_async_copy(
                bh.at[di.at[j]], bv.at[pl.ds(j * 128, 128)], sem).wait()

    def scat_fire(ov, di, sem):
        for j in range(8):
            pltpu.async_copy(ov.at[pl.ds(j * 128, 128)],
                             shared.at[di.at[j]], sem, add=True)

    def scat_drain(ov, di, sem):
        for j in range(8):
            pltpu.make_async_copy(ov.at[pl.ds(j * 128, 128)],
                                  shared.at[di.at[j]], sem).wait()

    def compute(av, bv, n128):
        def _ebody(i, acc):
            e = i * 4
            for u in range(4):
                x = jnp.maximum(av[e + u, 0:16] + bv[e + u, 0:16], 0.0)
                av[e + u, 0:16] = x
                acc = acc + x * x
            return acc
        acc = lax.fori_loop(0, n128 * 32, _ebody, zero16)
        s2_v[0:16] = s2_v[0:16] + acc

    idx_load(0, si0, di0)
    fire(si0, di0, a0, b0, gsem0)
    idx_load(1, si1, di1)
    fire(si1, di1, a1, b1, gsem1)
    plsc.subcore_barrier()

    def body(i, carry):
        g = 2 * i
        wait(si0, di0, a0, b0, gsem0)
        compute(a0, b0, 8)
        scat_fire(a0, di0, ssem0)

        wait(si1, di1, a1, b1, gsem1)
        compute(a1, b1, 8)
        scat_fire(a1, di1, ssem1)

        scat_drain(a0, di0, ssem0)
        idx_load(g + 2, si0, di0)
        fire(si0, di0, a0, b0, gsem0)

        scat_drain(a1, di1, ssem1)
        idx_load(g + 3, si1, di1)
        fire(si1, di1, a1, b1, gsem1)
        return carry

    lax.fori_loop(0, 23, body, 0)

    wait(si0, di0, a0, b0, gsem0)
    compute(a0, b0, 8)
    scat_fire(a0, di0, ssem0)
    wait(si1, di1, a1, b1, gsem1)
    compute(a1, b1, 8)
    scat_fire(a1, di1, ssem1)
    scat_drain(a0, di0, ssem0)
    scat_drain(a1, di1, ssem1)

    rem = jnp.where(s < 10, 391, 390) - 384

    def _tbody(t, carry):
        r = base + 384 + t
        sit = si0.at[0]
        dit = di0.at[0]
        avt = a0.at[pl.ds(0, 128)]
        bvt = b0.at[pl.ds(0, 128)]
        pltpu.sync_copy(src_hbm.at[pl.ds(r, 1)], si0.at[pl.ds(0, 1)])
        pltpu.sync_copy(dst_hbm.at[pl.ds(r, 1)], di0.at[pl.ds(0, 1)])
        pltpu.async_copy(ah.at[sit], avt, gsem0)
        pltpu.async_copy(bh.at[dit], bvt, gsem0)
        pltpu.make_async_copy(ah.at[sit], avt, gsem0).wait()
        pltpu.make_async_copy(bh.at[dit], bvt, gsem0).wait()
        compute(a0, b0, 1)
        pltpu.sync_copy(a0.at[pl.ds(0, 128)], shared.at[dit], add=True)
        return carry

    lax.fori_loop(0, rem, _tbody, 0)

    plsc.subcore_barrier()
    pltpu.sync_copy(shared.at[pl.ds(s * RPT, RPT)],
                    psum_out.at[c, pl.ds(s * RPT, RPT)])
    pltpu.sync_copy(s2_v, s2_out.at[c, s])


@functools.partial(
    pl.kernel,
    mesh=_mesh,
    out_type=[jax.ShapeDtypeStruct((32, NPAD), jnp.float32)],
    scratch_types=[
        pltpu.VMEM((8, 128), jnp.int32),
        pltpu.VMEM((NPAD,), jnp.float32),
    ],
    compiler_params=pltpu.CompilerParams(
        use_tc_tiling_on_sc=False, needs_layout_passes=False),
)
def _cnt_sc(dst_hbm, cnt_out, didx, hist):
    c = lax.axis_index("c")
    s = lax.axis_index("s")
    wid = s * 2 + c
    zero16 = jnp.zeros((16,), jnp.float32)
    one16 = zero16 + 1.0

    def _zh(i, carry):
        hist[pl.ds(i * 16, 16)] = zero16
        return carry
    lax.fori_loop(0, NPAD // 16, _zh, 0)

    def _process(row0, nr):
        pltpu.sync_copy(dst_hbm.at[pl.ds(row0, nr)], didx.at[pl.ds(0, nr)])
        for j in range(nr):
            for q in range(8):
                idx = didx[j, q * 16:(q + 1) * 16]
                plsc.addupdate_scatter(hist, [idx], one16)

    base = 195 * wid + jnp.minimum(wid, 10)

    def _gbody(g, carry):
        _process(base + g * 8, 8)
        return carry
    lax.fori_loop(0, 24, _gbody, 0)

    rem = jnp.where(wid < 10, 196, 195) - 192

    def _tbody(t, carry):
        _process(base + 192 + t, 1)
        return carry
    lax.fori_loop(0, rem, _tbody, 0)

    pltpu.sync_copy(hist, cnt_out.at[wid])


def _pre_body(f_ref, w1e_ref, b1e_ref, s_ref, wt_ref, wb_ref, bc_ref,
              a_ref, b_ref):
    y = jnp.maximum(
        jnp.dot(f_ref[...], w1e_ref[...], preferred_element_type=jnp.float32)
        + b1e_ref[...], 0.0)
    hid = jnp.dot(y, s_ref[...], preferred_element_type=jnp.float32)
    wt = wt_ref[...]
    wb = wb_ref[...]
    bc = bc_ref[...]
    a_ref[0] = jnp.dot(hid, wt[:, 0:16], preferred_element_type=jnp.float32)
    a_ref[1] = jnp.dot(hid, wt[:, 16:32], preferred_element_type=jnp.float32)
    b_ref[0] = jnp.dot(hid, wb[:, 0:16],
                       preferred_element_type=jnp.float32) + bc[:, 0:16]
    b_ref[1] = jnp.dot(hid, wb[:, 16:32],
                       preferred_element_type=jnp.float32) + bc[:, 16:32]


def _stats_body(pa_ref, pb_ref, s2_ref, sums_ref, mom_ref):
    i = pl.program_id(0)
    sblk = jnp.concatenate([pa_ref[0], pb_ref[0]], axis=1)
    sums_ref[...] = sblk
    col = jnp.sum(sblk, axis=0, keepdims=True)

    @pl.when(i == 0)
    def _():
        mom_ref[0:1, :] = col
        mom_ref[1:2, :] = jnp.concatenate(
            [jnp.sum(s2_ref[0], axis=0, keepdims=True),
             jnp.sum(s2_ref[1], axis=0, keepdims=True)], axis=1)

    @pl.when(i > 0)
    def _():
        mom_ref[0:1, :] = mom_ref[0:1, :] + col


def _cntred_body(cntp_ref, cnt_ref):
    ones = jnp.ones((32, 1), jnp.float32)
    cnt_ref[...] = lax.dot_general(
        cntp_ref[0], ones, (((0,), (0,)), ((), ())),
        preferred_element_type=jnp.float32)


def _bn_affine(mom, gamma, beta, denom_count):
    mu = mom[0:1, :] * (1.0 / denom_count)
    var = mom[1:2, :] * (1.0 / denom_count) - mu * mu
    a = gamma * lax.rsqrt(var + EPS)
    return a, beta - a * mu


def _upd1_body(sums_ref, cnt_ref, mom_ref, g_ref, be_ref, wt_ref, wb_ref,
               bc_ref, a_ref, b_ref):
    a, cadd = _bn_affine(mom_ref[...], g_ref[...], be_ref[...], E)
    cnt = cnt_ref[...]
    hid = (a * sums_ref[...] + cadd * cnt) / jnp.maximum(cnt, 1.0)
    wt = wt_ref[...]
    wb = wb_ref[...]
    bc = bc_ref[...]
    a_ref[0] = jnp.dot(hid, wt[:, 0:16], preferred_element_type=jnp.float32)
    a_ref[1] = jnp.dot(hid, wt[:, 16:32], preferred_element_type=jnp.float32)
    b_ref[0] = jnp.dot(hid, wb[:, 0:16],
                       preferred_element_type=jnp.float32) + bc[:, 0:16]
    b_ref[1] = jnp.dot(hid, wb[:, 16:32],
                       preferred_element_type=jnp.float32) + bc[:, 16:32]


def _upd2_body(sums_ref, cnt_ref, mom_ref, g_ref, be_ref, wm_ref, bm_ref,
               z_ref, zmom_ref):
    i = pl.program_id(0)
    a, cadd = _bn_affine(mom_ref[...], g_ref[...], be_ref[...], E)
    cnt = cnt_ref[...]
    hid = (a * sums_ref[...] + cadd * cnt) / jnp.maximum(cnt, 1.0)
    z = jnp.maximum(
        jnp.dot(hid, wm_ref[...], preferred_element_type=jnp.float32)
        + bm_ref[...], 0.0)
    z_ref[...] = z
    col = jnp.sum(z, axis=0, keepdims=True)
    col2 = jnp.sum(z * z, axis=0, keepdims=True)

    @pl.when(i == 0)
    def _():
        zmom_ref[0:1, :] = col
        zmom_ref[1:2, :] = col2

    @pl.when(i > 0)
    def _():
        zmom_ref[0:1, :] = zmom_ref[0:1, :] + col
        zmom_ref[1:2, :] = zmom_ref[1:2, :] + col2


def _final_body(z_ref, zmom_ref, g_ref, be_ref, wo_ref, bo_ref, out_ref):
    a, cadd = _bn_affine(zmom_ref[...], g_ref[...], be_ref[...], N)
    h2 = a * z_ref[...] + cadd
    logit = jnp.sum(h2 * wo_ref[...], axis=1, keepdims=True) + bo_ref[...]
    out_ref[...] = jax.nn.sigmoid(logit)


def _blk(shape):
    return pl.BlockSpec(shape, lambda i: (i,) + (0,) * (len(shape) - 1))


def _blk2(shape):
    return pl.BlockSpec(shape, lambda i: (0, i, 0))


def _fix(shape):
    return pl.BlockSpec(shape, lambda i: (0,) * len(shape))


def kernel(feat, edge_index, W1, b1, Wc1, bc1, gc1, bec1, Wc2, bc2, gc2, bec2,
           Wm, bm, gm, bem, Wo, bo):
    feat2 = feat.reshape(N, 16)
    src2 = edge_index[0].reshape(NROWS, 128)
    dst2 = edge_index[1].reshape(NROWS, 128)
    W1e = jnp.kron(jnp.eye(8, dtype=jnp.float32), W1)
    b1e = jnp.tile(b1, 8).reshape(1, 256)
    S8 = jnp.tile(jnp.eye(32, dtype=jnp.float32) * 0.125, (8, 1))

    A1, B1 = pl.pallas_call(
        _pre_body,
        grid=(GRID,),
        in_specs=[_blk((BN, 16)), _fix((16, 256)), _fix((1, 256)),
                  _fix((256, 32)), _fix((32, 32)), _fix((32, 32)),
                  _fix((1, 32))],
        out_specs=[_blk2((2, BN, 16)), _blk2((2, BN, 16))],
        out_shape=[jax.ShapeDtypeStruct((2, N, 16), jnp.float32)] * 2,
    )(feat2, W1e, b1e, S8, Wc1[:32], Wc1[32:], bc1.reshape(1, 32))

    cntp = _cnt_sc(dst2)[0]
    cntp3 = cntp[:, :N].reshape(32, GRID, BN).transpose(1, 0, 2)
    cntc = pl.pallas_call(
        _cntred_body,
        grid=(GRID,),
        in_specs=[pl.BlockSpec((1, 32, BN), lambda i: (i, 0, 0))],
        out_specs=_blk((BN, 1)),
        out_shape=jax.ShapeDtypeStruct((N, 1), jnp.float32),
    )(cntp3)

    def edge_layer(A, B):
        p, s2 = _edge_sc(A, B, src2, dst2)
        return pl.pallas_call(
            _stats_body,
            grid=(GRID,),
            in_specs=[_blk2((1, BN, 16)),
                      pl.BlockSpec((1, BN, 16), lambda i: (1, i, 0)),
                      _fix((2, 16, 16))],
            out_specs=[_blk((BN, 32)), _fix((2, 32))],
            out_shape=[jax.ShapeDtypeStruct((N, 32), jnp.float32),
                       jax.ShapeDtypeStruct((2, 32), jnp.float32)],
        )(p, p, s2)

    sums1, mom1 = edge_layer(A1, B1)

    A2, B2 = pl.pallas_call(
        _upd1_body,
        grid=(GRID,),
        in_specs=[_blk((BN, 32)), _blk((BN, 1)), _fix((2, 32)),
                  _fix((1, 32)), _fix((1, 32)), _fix((32, 32)),
                  _fix((32, 32)), _fix((1, 32))],
        out_specs=[_blk2((2, BN, 16)), _blk2((2, BN, 16))],
        out_shape=[jax.ShapeDtypeStruct((2, N, 16), jnp.float32)] * 2,
    )(sums1, cntc, mom1, gc1.reshape(1, 32), bec1.reshape(1, 32),
      Wc2[:32], Wc2[32:], bc2.reshape(1, 32))

    sums2, mom2 = edge_layer(A2, B2)

    z, zmom = pl.pallas_call(
        _upd2_body,
        grid=(GRID,),
        in_specs=[_blk((BN, 32)), _blk((BN, 1)), _fix((2, 32)),
                  _fix((1, 32)), _fix((1, 32)), _fix((32, 16)),
                  _fix((1, 16))],
        out_specs=[_blk((BN, 16)), _fix((2, 16))],
        out_shape=[jax.ShapeDtypeStruct((N, 16), jnp.float32),
                   jax.ShapeDtypeStruct((2, 16), jnp.float32)],
    )(sums2, cntc, mom2, gc2.reshape(1, 32), bec2.reshape(1, 32),
      Wm, bm.reshape(1, 16))

    out = pl.pallas_call(
        _final_body,
        grid=(GRID,),
        in_specs=[_blk((BN, 16)), _fix((2, 16)), _fix((1, 16)),
                  _fix((1, 16)), _fix((1, 16)), _fix((1, 1))],
        out_specs=_blk((BN, 1)),
        out_shape=jax.ShapeDtypeStruct((N, 1), jnp.float32),
    )(z, zmom, gm.reshape(1, 16), bem.reshape(1, 16),
      Wo.reshape(1, 16), bo.reshape(1, 1))

    return out


# speedup vs baseline: 13.9149x; 1.0780x over previous
import functools

import jax
import jax.numpy as jnp
from jax import lax
from jax.experimental import pallas as pl
from jax.experimental.pallas import tpu as pltpu
from jax.experimental.pallas import tpu_sc as plsc

N = 50000
E = 800000
EPS = 1e-5
NROWS = E // 128
NPAD = 50176
RPT = NPAD // 16
ZR = RPT // 8
BN = 5000
GRID = N // BN

_mesh = plsc.VectorSubcoreMesh(core_axis_name="c", subcore_axis_name="s")
_sc_params = pltpu.CompilerParams(use_tc_tiling_on_sc=False)


@functools.partial(
    pl.kernel,
    mesh=_mesh,
    out_type=[
        jax.ShapeDtypeStruct((2, NPAD, 16), jnp.float32),
        jax.ShapeDtypeStruct((2, 16, 16), jnp.float32),
    ],
    scratch_types=[
        pltpu.VMEM((8, 128), jnp.int32),
        pltpu.VMEM((8, 128), jnp.int32),
        pltpu.VMEM((8, 128), jnp.int32),
        pltpu.VMEM((8, 128), jnp.int32),
        pltpu.VMEM((8, 128), jnp.int32),
        pltpu.VMEM((8, 128), jnp.int32),
        pltpu.VMEM((1024, 16), jnp.float32),
        pltpu.VMEM((1024, 16), jnp.float32),
        pltpu.VMEM((1024, 16), jnp.float32),
        pltpu.VMEM((1024, 16), jnp.float32),
        pltpu.VMEM((16,), jnp.float32),
        pltpu.VMEM_SHARED((NPAD, 16), jnp.float32),
        pltpu.SemaphoreType.DMA,
        pltpu.SemaphoreType.DMA,
        pltpu.SemaphoreType.DMA,
        pltpu.SemaphoreType.DMA,
    ],
    compiler_params=_sc_params,
)
def _edge_sc(t_hbm, src_hbm, dst4_hbm, dst_hbm, psum_out, s2_out,
             si0, gi0, di0, si1, gi1, di1, a0, b0, a1, b1,
             s2_v, shared, gsem0, gsem1, ssem0, ssem1):
    c = lax.axis_index("c")
    s = lax.axis_index("s")
    zero16 = jnp.zeros((16,), jnp.float32)
    tv = t_hbm.at[pl.ds(c, 4 * N - 1)]

    def _zb(r, carry):
        a0[r, 0:16] = zero16
        return carry
    lax.fori_loop(0, ZR, _zb, 0)
    for t in range(8):
        pltpu.sync_copy(a0.at[pl.ds(0, ZR)],
                        shared.at[pl.ds(s * RPT + t * ZR, ZR)])
    s2_v[0:16] = zero16

    base = 390 * s + jnp.minimum(s, 10)

    def idx_load(g, si, gi, di):
        r = base + g * 8
        pltpu.sync_copy(src_hbm.at[pl.ds(r, 8)], si)
        pltpu.sync_copy(dst4_hbm.at[pl.ds(r, 8)], gi)
        pltpu.sync_copy(dst_hbm.at[pl.ds(r, 8)], di)

    def fire(si, gi, av, bv, sem):
        for j in range(8):
            pltpu.async_copy(tv.at[si.at[j]], av.at[pl.ds(j * 128, 128)], sem)
            pltpu.async_copy(tv.at[gi.at[j]], bv.at[pl.ds(j * 128, 128)], sem)

    def wait(si, gi, av, bv, sem):
        for j in range(8):
            pltpu.make_async_copy(
                tv.at[si.at[j]], av.at[pl.ds(j * 128, 128)], sem).wait()
            pltpu.make_async_copy(
                tv.at[gi.at[j]], bv.at[pl.ds(j * 128, 128)], sem).wait()

    def scat_fire(ov, di, sem):
        for j in range(8):
            pltpu.async_copy(ov.at[pl.ds(j * 128, 128)],
                             shared.at[di.at[j]], sem, add=True)

    def scat_drain(ov, di, sem):
        for j in range(8):
            pltpu.make_async_copy(ov.at[pl.ds(j * 128, 128)],
                                  shared.at[di.at[j]], sem).wait()

    def compute(av, bv, n128):
        def _ebody(i, acc):
            e = i * 4
            sq = []
            for u in range(4):
                x = jnp.maximum(av[e + u, 0:16] + bv[e + u, 0:16], 0.0)
                av[e + u, 0:16] = x
                sq.append(x * x)
            return acc + ((sq[0] + sq[1]) + (sq[2] + sq[3]))
        acc = lax.fori_loop(0, n128 * 32, _ebody, zero16)
        s2_v[0:16] = s2_v[0:16] + acc

    idx_load(0, si0, gi0, di0)
    fire(si0, gi0, a0, b0, gsem0)
    idx_load(1, si1, gi1, di1)
    fire(si1, gi1, a1, b1, gsem1)
    plsc.subcore_barrier()

    def body(i, carry):
        g = 2 * i
        wait(si0, gi0, a0, b0, gsem0)
        compute(a0, b0, 8)
        scat_fire(a0, di0, ssem0)

        wait(si1, gi1, a1, b1, gsem1)
        compute(a1, b1, 8)
        scat_fire(a1, di1, ssem1)

        scat_drain(a0, di0, ssem0)
        idx_load(g + 2, si0, gi0, di0)
        fire(si0, gi0, a0, b0, gsem0)

        scat_drain(a1, di1, ssem1)
        idx_load(g + 3, si1, gi1, di1)
        fire(si1, gi1, a1, b1, gsem1)
        return carry

    lax.fori_loop(0, 23, body, 0)

    wait(si0, gi0, a0, b0, gsem0)
    compute(a0, b0, 8)
    scat_fire(a0, di0, ssem0)
    wait(si1, gi1, a1, b1, gsem1)
    compute(a1, b1, 8)
    scat_fire(a1, di1, ssem1)
    scat_drain(a0, di0, ssem0)
    scat_drain(a1, di1, ssem1)

    rem = jnp.where(s < 10, 391, 390) - 384

    def _tbody(t, carry):
        r = base + 384 + t
        sit = si0.at[0]
        git = gi0.at[0]
        dit = di0.at[0]
        avt = a0.at[pl.ds(0, 128)]
        bvt = b0.at[pl.ds(0, 128)]
        pltpu.sync_copy(src_hbm.at[pl.ds(r, 1)], si0.at[pl.ds(0, 1)])
        pltpu.sync_copy(dst4_hbm.at[pl.ds(r, 1)], gi0.at[pl.ds(0, 1)])
        pltpu.sync_copy(dst_hbm.at[pl.ds(r, 1)], di0.at[pl.ds(0, 1)])
        w1 = pltpu.async_copy(tv.at[sit], avt, gsem0)
        w2 = pltpu.async_copy(tv.at[git], bvt, gsem0)
        w1.wait()
        w2.wait()
        compute(a0, b0, 1)
        pltpu.sync_copy(a0.at[pl.ds(0, 128)], shared.at[dit], add=True)
        return carry

    lax.fori_loop(0, rem, _tbody, 0)

    plsc.subcore_barrier()
    pltpu.sync_copy(shared.at[pl.ds(s * RPT, RPT)],
                    psum_out.at[c, pl.ds(s * RPT, RPT)])
    pltpu.sync_copy(s2_v, s2_out.at[c, s])


@functools.partial(
    pl.kernel,
    mesh=_mesh,
    out_type=[jax.ShapeDtypeStruct((32, NPAD), jnp.float32)],
    scratch_types=[
        pltpu.VMEM((8, 128), jnp.int32),
        pltpu.VMEM((NPAD,), jnp.float32),
    ],
    compiler_params=pltpu.CompilerParams(
        use_tc_tiling_on_sc=False, needs_layout_passes=False),
)
def _cnt_sc(dst_hbm, cnt_out, didx, hist):
    c = lax.axis_index("c")
    s = lax.axis_index("s")
    wid = s * 2 + c
    zero16 = jnp.zeros((16,), jnp.float32)
    one16 = zero16 + 1.0

    def _zh(i, carry):
        hist[pl.ds(i * 16, 16)] = zero16
        return carry
    lax.fori_loop(0, NPAD // 16, _zh, 0)

    def _process(row0, nr):
        pltpu.sync_copy(dst_hbm.at[pl.ds(row0, nr)], didx.at[pl.ds(0, nr)])
        for j in range(nr):
            for q in range(8):
                idx = didx[j, q * 16:(q + 1) * 16]
                plsc.addupdate_scatter(hist, [idx], one16)

    base = 195 * wid + jnp.minimum(wid, 10)

    def _gbody(g, carry):
        _process(base + g * 8, 8)
        return carry
    lax.fori_loop(0, 24, _gbody, 0)

    rem = jnp.where(wid < 10, 196, 195) - 192

    def _tbody(t, carry):
        _process(base + 192 + t, 1)
        return carry
    lax.fori_loop(0, rem, _tbody, 0)

    pltpu.sync_copy(hist, cnt_out.at[wid])


def _pre_body(f_ref, w1e_ref, b1e_ref, s_ref, wt_ref, wb_ref, bc_ref, ab_ref):
    y = jnp.maximum(
        jnp.dot(f_ref[...], w1e_ref[...], preferred_element_type=jnp.float32)
        + b1e_ref[...], 0.0)
    hid = jnp.dot(y, s_ref[...], preferred_element_type=jnp.float32)
    ab_ref[...] = jnp.concatenate(
        [jnp.dot(hid, wt_ref[...], preferred_element_type=jnp.float32),
         jnp.dot(hid, wb_ref[...], preferred_element_type=jnp.float32)
         + bc_ref[...]], axis=1)


def _stats_body(pa_ref, pb_ref, s2_ref, sums_ref, mom_ref):
    i = pl.program_id(0)
    sblk = jnp.concatenate([pa_ref[0], pb_ref[0]], axis=1)
    sums_ref[...] = sblk
    col = jnp.sum(sblk, axis=0, keepdims=True)

    @pl.when(i == 0)
    def _():
        mom_ref[0:1, :] = col
        mom_ref[1:2, :] = jnp.concatenate(
            [jnp.sum(s2_ref[0], axis=0, keepdims=True),
             jnp.sum(s2_ref[1], axis=0, keepdims=True)], axis=1)

    @pl.when(i > 0)
    def _():
        mom_ref[0:1, :] = mom_ref[0:1, :] + col


def _cntred_body(cntp_ref, cnt_ref):
    ones = jnp.ones((32, 1), jnp.float32)
    cnt_ref[...] = lax.dot_general(
        cntp_ref[0], ones, (((0,), (0,)), ((), ())),
        preferred_element_type=jnp.float32)


def _bn_affine(mom, gamma, beta, denom_count):
    mu = mom[0:1, :] * (1.0 / denom_count)
    var = mom[1:2, :] * (1.0 / denom_count) - mu * mu
    a = gamma * lax.rsqrt(var + EPS)
    return a, beta - a * mu


def _upd1_body(sums_ref, cnt_ref, mom_ref, g_ref, be_ref, wt_ref, wb_ref,
               bc_ref, ab_ref):
    a, cadd = _bn_affine(mom_ref[...], g_ref[...], be_ref[...], E)
    cnt = cnt_ref[...]
    hid = (a * sums_ref[...] + cadd * cnt) / jnp.maximum(cnt, 1.0)
    ab_ref[...] = jnp.concatenate(
        [jnp.dot(hid, wt_ref[...], preferred_element_type=jnp.float32),
         jnp.dot(hid, wb_ref[...], preferred_element_type=jnp.float32)
         + bc_ref[...]], axis=1)


def _upd2_body(sums_ref, cnt_ref, mom_ref, g_ref, be_ref, wm_ref, bm_ref,
               z_ref, zmom_ref):
    i = pl.program_id(0)
    a, cadd = _bn_affine(mom_ref[...], g_ref[...], be_ref[...], E)
    cnt = cnt_ref[...]
    hid = (a * sums_ref[...] + cadd * cnt) / jnp.maximum(cnt, 1.0)
    z = jnp.maximum(
        jnp.dot(hid, wm_ref[...], preferred_element_type=jnp.float32)
        + bm_ref[...], 0.0)
    z_ref[...] = z
    col = jnp.sum(z, axis=0, keepdims=True)
    col2 = jnp.sum(z * z, axis=0, keepdims=True)

    @pl.when(i == 0)
    def _():
        zmom_ref[0:1, :] = col
        zmom_ref[1:2, :] = col2

    @pl.when(i > 0)
    def _():
        zmom_ref[0:1, :] = zmom_ref[0:1, :] + col
        zmom_ref[1:2, :] = zmom_ref[1:2, :] + col2


def _final_body(z_ref, zmom_ref, g_ref, be_ref, wo_ref, bo_ref, out_ref):
    a, cadd = _bn_affine(zmom_ref[...], g_ref[...], be_ref[...], N)
    h2 = a * z_ref[...] + cadd
    logit = jnp.sum(h2 * wo_ref[...], axis=1, keepdims=True) + bo_ref[...]
    out_ref[...] = jax.nn.sigmoid(logit)


def _blk(shape):
    return pl.BlockSpec(shape, lambda i: (i,) + (0,) * (len(shape) - 1))


def _blk2(shape):
    return pl.BlockSpec(shape, lambda i: (0, i, 0))


def _fix(shape):
    return pl.BlockSpec(shape, lambda i: (0,) * len(shape))


def kernel(feat, edge_index, W1, b1, Wc1, bc1, gc1, bec1, Wc2, bc2, gc2, bec2,
           Wm, bm, gm, bem, Wo, bo):
    feat2 = feat.reshape(N, 16)
    src2 = edge_index[0].reshape(NROWS, 128)
    dst2 = edge_index[1].reshape(NROWS, 128)
    src4 = src2 * 4
    dst4 = dst2 * 4 + 2
    W1e = jnp.kron(jnp.eye(8, dtype=jnp.float32), W1)
    b1e = jnp.tile(b1, 8).reshape(1, 256)
    S8 = jnp.tile(jnp.eye(32, dtype=jnp.float32) * 0.125, (8, 1))

    AB1 = pl.pallas_call(
        _pre_body,
        grid=(GRID,),
        in_specs=[_blk((BN, 16)), _fix((16, 256)), _fix((1, 256)),
                  _fix((256, 32)), _fix((32, 32)), _fix((32, 32)),
                  _fix((1, 32))],
        out_specs=_blk((BN, 64)),
        out_shape=jax.ShapeDtypeStruct((N, 64), jnp.float32),
    )(feat2, W1e, b1e, S8, Wc1[:32], Wc1[32:], bc1.reshape(1, 32))

    cntp = _cnt_sc(dst2)[0]
    cntp3 = cntp[:, :N].reshape(32, GRID, BN).transpose(1, 0, 2)
    cntc = pl.pallas_call(
        _cntred_body,
        grid=(GRID,),
        in_specs=[pl.BlockSpec((1, 32, BN), lambda i: (i, 0, 0))],
        out_specs=_blk((BN, 1)),
        out_shape=jax.ShapeDtypeStruct((N, 1), jnp.float32),
    )(cntp3)

    def edge_layer(AB):
        p, s2 = _edge_sc(AB.reshape(4 * N, 16), src4, dst4, dst2)
        return pl.pallas_call(
            _stats_body,
            grid=(GRID,),
            in_specs=[_blk2((1, BN, 16)),
                      pl.BlockSpec((1, BN, 16), lambda i: (1, i, 0)),
                      _fix((2, 16, 16))],
            out_specs=[_blk((BN, 32)), _fix((2, 32))],
            out_shape=[jax.ShapeDtypeStruct((N, 32), jnp.float32),
                       jax.ShapeDtypeStruct((2, 32), jnp.float32)],
        )(p, p, s2)

    sums1, mom1 = edge_layer(AB1)

    AB2 = pl.pallas_call(
        _upd1_body,
        grid=(GRID,),
        in_specs=[_blk((BN, 32)), _blk((BN, 1)), _fix((2, 32)),
                  _fix((1, 32)), _fix((1, 32)), _fix((32, 32)),
                  _fix((32, 32)), _fix((1, 32))],
        out_specs=_blk((BN, 64)),
        out_shape=jax.ShapeDtypeStruct((N, 64), jnp.float32),
    )(sums1, cntc, mom1, gc1.reshape(1, 32), bec1.reshape(1, 32),
      Wc2[:32], Wc2[32:], bc2.reshape(1, 32))

    sums2, mom2 = edge_layer(AB2)

    z, zmom = pl.pallas_call(
        _upd2_body,
        grid=(GRID,),
        in_specs=[_blk((BN, 32)), _blk((BN, 1)), _fix((2, 32)),
                  _fix((1, 32)), _fix((1, 32)), _fix((32, 16)),
                  _fix((1, 16))],
        out_specs=[_blk((BN, 16)), _fix((2, 16))],
        out_shape=[jax.ShapeDtypeStruct((N, 16), jnp.float32),
                   jax.ShapeDtypeStruct((2, 16), jnp.float32)],
    )(sums2, cntc, mom2, gc2.reshape(1, 32), bec2.reshape(1, 32),
      Wm, bm.reshape(1, 16))

    out = pl.pallas_call(
        _final_body,
        grid=(GRID,),
        in_specs=[_blk((BN, 16)), _fix((2, 16)), _fix((1, 16)),
                  _fix((1, 16)), _fix((1, 16)), _fix((1, 1))],
        out_specs=_blk((BN, 1)),
        out_shape=jax.ShapeDtypeStruct((N, 1), jnp.float32),
    )(z, zmom, gm.reshape(1, 16), bem.reshape(1, 16),
      Wo.reshape(1, 16), bo.reshape(1, 1))

    return out
